# Initial kernel scaffold; baseline (speedup 1.0000x reference)
import jax
import jax.numpy as jnp
from jax.experimental import pallas as pl


def kernel(x, edge_index, W_self0, W_neigh0, b0, W_self1, W_neigh1, b1):
    raise NotImplementedError("write your pallas kernel here")


# speedup vs baseline: 5.5224x; 5.5224x over previous
import jax
import jax.numpy as jnp
from jax import lax
from jax.experimental import pallas as pl
from jax.experimental.pallas import tpu as pltpu
from jax.experimental.pallas import tpu_sc as plsc

_N = 10000
_E = 160000
_IN = 256
_HID = 512
_OUT = 256
_F = 128
_NC, _NS = 2, 16
_CH = 128
_NCHUNK = _E // _CH
_CPS = -(-_NCHUNK // _NS)
_NP = 10240
_RPS = _NP // _NS
_ZR = 128
_BM = 400


def _make_sc_agg(with_deg):
  mesh = plsc.VectorSubcoreMesh(core_axis_name="c", subcore_axis_name="s")
  out_type = [jax.ShapeDtypeStruct((_NC, _NP, _F), jnp.float32)]
  scratch = [
      pltpu.VMEM_SHARED((_NP, _F), jnp.float32),
      pltpu.VMEM((_ZR, _F), jnp.float32),
      pltpu.VMEM((_CH,), jnp.int32),
      pltpu.VMEM((_CH,), jnp.int32),
      pltpu.VMEM((_CH, _F), jnp.float32),
      pltpu.SemaphoreType.DMA,
  ]
  if with_deg:
    out_type.append(jax.ShapeDtypeStruct((_NP,), jnp.float32))
    scratch += [
        pltpu.VMEM_SHARED((_NP,), jnp.float32),
        pltpu.VMEM((_RPS,), jnp.float32),
        pltpu.VMEM((_CH,), jnp.float32),
    ]

  def body(rows2n, srcs, dst, *rest):
    if with_deg:
      out, deg_out = rest[0], rest[1]
      acc_sh, zbuf, idxs_v, idxd_v, rows_v, sem, deg_sh, zdeg, ones_v = rest[2:]
    else:
      out = rest[0]
      acc_sh, zbuf, idxs_v, idxd_v, rows_v, sem = rest[1:]
    c = lax.axis_index("c")
    s = lax.axis_index("s")
    zero16 = jnp.zeros((16,), jnp.float32)

    def zb(i, carry):
      for j in range(_F // 16):
        zbuf[i, pl.ds(j * 16, 16)] = zero16
      return carry
    lax.fori_loop(0, _ZR, zb, 0)

    r0 = s * _RPS
    for k in range(_RPS // _ZR):
      pltpu.sync_copy(zbuf, acc_sh.at[pl.ds(r0 + k * _ZR, _ZR)])

    if with_deg:
      @pl.when(c == 0)
      def _init_deg():
        def zd(i, carry):
          zdeg[pl.ds(i * 16, 16)] = zero16
          return carry
        lax.fori_loop(0, _RPS // 16, zd, 0)
        pltpu.sync_copy(zdeg, deg_sh.at[pl.ds(r0, _RPS)])
        one16 = jnp.full((16,), 1.0, jnp.float32)
        for j in range(_CH // 16):
          ones_v[pl.ds(j * 16, 16)] = one16

    plsc.subcore_barrier()

    def chunk(i, carry):
      k = s + i * _NS
      @pl.when(k < _NCHUNK)
      def _do():
        e0 = k * _CH
        pltpu.sync_copy(srcs.at[c, pl.ds(e0, _CH)], idxs_v)
        pltpu.sync_copy(dst.at[pl.ds(e0, _CH)], idxd_v)
        pltpu.async_copy(rows2n.at[idxs_v], rows_v, sem).wait()
        pltpu.sync_copy(rows_v, acc_sh.at[idxd_v], add=True)
        if with_deg:
          @pl.when(c == 0)
          def _deg():
            pltpu.sync_copy(ones_v, deg_sh.at[idxd_v], add=True)
      return carry
    lax.fori_loop(0, _CPS, chunk, 0)

    plsc.subcore_barrier()

    for k in range(_RPS // _ZR):
      rr = r0 + k * _ZR
      pltpu.sync_copy(acc_sh.at[pl.ds(rr, _ZR)], zbuf)
      pltpu.sync_copy(zbuf, out.at[c, pl.ds(rr, _ZR)])
    if with_deg:
      @pl.when(c == 0)
      def _wb_deg():
        pltpu.sync_copy(deg_sh.at[pl.ds(r0, _RPS)], zdeg)
        pltpu.sync_copy(zdeg, deg_out.at[pl.ds(r0, _RPS)])

  return pl.kernel(body, out_type=tuple(out_type), mesh=mesh,
                   scratch_types=tuple(scratch))


_sc_agg_deg = _make_sc_agg(True)
_sc_agg = _make_sc_agg(False)


def _l0_body(x_ref, agg_ref, deg_ref, ws_ref, wn_ref, b_ref, o_ref):
  inv = 1.0 / jnp.maximum(deg_ref[...], 1.0)
  acc = jnp.dot(x_ref[...], ws_ref[...], preferred_element_type=jnp.float32)
  acc += jnp.dot(agg_ref[0] * inv, wn_ref[0:_F, :],
                 preferred_element_type=jnp.float32)
  acc += jnp.dot(agg_ref[1] * inv, wn_ref[_F:2 * _F, :],
                 preferred_element_type=jnp.float32)
  o_ref[...] = jnp.maximum(acc + b_ref[...], 0.0)


def _l1a_body(h_ref, ws_ref, wn_ref, b_ref, part_ref, p_ref):
  h = h_ref[...]
  part_ref[...] = (
      jnp.dot(h, ws_ref[...], preferred_element_type=jnp.float32) + b_ref[...])
  p = jnp.dot(h, wn_ref[...], preferred_element_type=jnp.float32)
  p_ref[:, 0, :] = p[:, 0:_F]
  p_ref[:, 1, :] = p[:, _F:2 * _F]


def _l1b_body(part_ref, agg_ref, deg_ref, o_ref):
  inv = 1.0 / jnp.maximum(deg_ref[...], 1.0)
  o_ref[:, 0:_F] = part_ref[:, 0:_F] + agg_ref[0] * inv
  o_ref[:, _F:2 * _F] = part_ref[:, _F:2 * _F] + agg_ref[1] * inv


def _l0(x, agg0, deg, ws, wn, b):
  return pl.pallas_call(
      _l0_body,
      grid=(_N // _BM,),
      in_specs=[
          pl.BlockSpec((_BM, _IN), lambda i: (i, 0)),
          pl.BlockSpec((_NC, _BM, _F), lambda i: (0, i, 0)),
          pl.BlockSpec((_BM, 1), lambda i: (i, 0)),
          pl.BlockSpec((_IN, _HID), lambda i: (0, 0)),
          pl.BlockSpec((_IN, _HID), lambda i: (0, 0)),
          pl.BlockSpec((1, _HID), lambda i: (0, 0)),
      ],
      out_specs=pl.BlockSpec((_BM, _HID), lambda i: (i, 0)),
      out_shape=jax.ShapeDtypeStruct((_N, _HID), jnp.float32),
  )(x, agg0, deg, ws, wn, b)


def _l1a(h, ws, wn, b):
  return pl.pallas_call(
      _l1a_body,
      grid=(_N // _BM,),
      in_specs=[
          pl.BlockSpec((_BM, _HID), lambda i: (i, 0)),
          pl.BlockSpec((_HID, _OUT), lambda i: (0, 0)),
          pl.BlockSpec((_HID, _OUT), lambda i: (0, 0)),
          pl.BlockSpec((1, _OUT), lambda i: (0, 0)),
      ],
      out_specs=[
          pl.BlockSpec((_BM, _OUT), lambda i: (i, 0)),
          pl.BlockSpec((_BM, _NC, _F), lambda i: (i, 0, 0)),
      ],
      out_shape=[
          jax.ShapeDtypeStruct((_N, _OUT), jnp.float32),
          jax.ShapeDtypeStruct((_N, _NC, _F), jnp.float32),
      ],
  )(h, ws, wn, b)


def _l1b(part, agg1, deg):
  return pl.pallas_call(
      _l1b_body,
      grid=(_N // _BM,),
      in_specs=[
          pl.BlockSpec((_BM, _OUT), lambda i: (i, 0)),
          pl.BlockSpec((_NC, _BM, _F), lambda i: (0, i, 0)),
          pl.BlockSpec((_BM, 1), lambda i: (i, 0)),
      ],
      out_specs=pl.BlockSpec((_BM, _OUT), lambda i: (i, 0)),
      out_shape=jax.ShapeDtypeStruct((_N, _OUT), jnp.float32),
  )(part, agg1, deg)


def kernel(x, edge_index, W_self0, W_neigh0, b0, W_self1, W_neigh1, b1):
  ei = edge_index.astype(jnp.int32)
  src, dst = ei[0], ei[1]
  srcs = jnp.stack([2 * src, 2 * src + 1])
  agg0, deg1 = _sc_agg_deg(x.reshape(2 * _N, _F), srcs, dst)
  deg = deg1.reshape(_NP, 1)
  h = _l0(x, agg0, deg, W_self0, W_neigh0, b0.reshape(1, _HID))
  part, p2 = _l1a(h, W_self1, W_neigh1, b1.reshape(1, _OUT))
  (agg1,) = _sc_agg(p2.reshape(2 * _N, _F), srcs, dst)
  out = _l1b(part, agg1, deg)
  return out
